# Initial kernel scaffold; baseline (speedup 1.0000x reference)
import jax
import jax.numpy as jnp
from jax.experimental import pallas as pl


def kernel(x, edge_index, hyperedge_attr, W_lin, b_lin, W_sheaf, b_sheaf, W_conv1, b_conv1, W_conv2, b_conv2, W_lin2):
    raise NotImplementedError("write your pallas kernel here")


# speedup vs baseline: 3.8411x; 3.8411x over previous
import functools

import jax
import jax.numpy as jnp
import numpy as np
from jax import lax
from jax.experimental import pallas as pl
from jax.experimental.pallas import tpu as pltpu
from jax.experimental.pallas import tpu_sc as plsc

D = 6
F = 64
DF = D * F
HALF = DF // 2
N = 10000
E = 10000
NPAD = 10240
BLK = 1024
NNZ = 160000
NFEAT = 256
NCLS = 40
LANE_PAD = 16


def _safe_inv_sqrt(v):
    vs = jnp.where(v > 0, v, 1.0)
    return jnp.where(v > 0, lax.rsqrt(vs), 0.0)


def _safe_inv(v):
    vs = jnp.where(v > 0, v, 1.0)
    return jnp.where(v > 0, 1.0 / vs, 0.0)


def _dense1_body(x_ref, he_ref, Wl_ref, bl_ref, Ws_ref, bs_ref,
                 xs0_ref, xs1_ref, px_ref, pe_ref):
    Wl = Wl_ref[...]
    bl = bl_ref[...]
    Wm = sum(Wl[:, j * F:(j + 1) * F] for j in range(D)) * (1.0 / D)
    bm = sum(bl[:, j * F:(j + 1) * F] for j in range(D)) * (1.0 / D)
    A0 = Wm @ Ws_ref[:F, :]
    A1 = Wm @ Ws_ref[F:, :]
    b0 = bm @ Ws_ref[:F, :]
    b1 = bm @ Ws_ref[F:, :] + bs_ref[...]

    xl = x_ref[...] @ Wl + bl
    xs0_ref[...] = xl[:, :HALF]
    xs1_ref[...] = xl[:, HALF:]

    pxv = x_ref[...] @ A0 + b0
    pev = he_ref[...] @ A1 + b1
    zeros = jnp.zeros((pxv.shape[0], LANE_PAD - D), jnp.float32)
    px_ref[...] = jnp.concatenate([pxv, zeros], axis=1)
    pe_ref[...] = jnp.concatenate([pev, zeros], axis=1)


def _dense1(x, he, W_lin, b_lin, W_sheaf, b_sheaf):
    blk = BLK
    grid = NPAD // blk
    return pl.pallas_call(
        _dense1_body,
        grid=(grid,),
        in_specs=[
            pl.BlockSpec((blk, NFEAT), lambda i: (i, 0)),
            pl.BlockSpec((blk, NFEAT), lambda i: (i, 0)),
            pl.BlockSpec((NFEAT, DF), lambda i: (0, 0)),
            pl.BlockSpec((1, DF), lambda i: (0, 0)),
            pl.BlockSpec((2 * F, D), lambda i: (0, 0)),
            pl.BlockSpec((1, D), lambda i: (0, 0)),
        ],
        out_specs=[
            pl.BlockSpec((blk, HALF), lambda i: (i, 0)),
            pl.BlockSpec((blk, HALF), lambda i: (i, 0)),
            pl.BlockSpec((blk, LANE_PAD), lambda i: (i, 0)),
            pl.BlockSpec((blk, LANE_PAD), lambda i: (i, 0)),
        ],
        out_shape=[
            jax.ShapeDtypeStruct((NPAD, HALF), jnp.float32),
            jax.ShapeDtypeStruct((NPAD, HALF), jnp.float32),
            jax.ShapeDtypeStruct((NPAD, LANE_PAD), jnp.float32),
            jax.ShapeDtypeStruct((NPAD, LANE_PAD), jnp.float32),
        ],
    )(x, he, W_lin, b_lin.reshape(1, DF), W_sheaf, b_sheaf.reshape(1, D))


def _norms_body(Dv_ref, Be_ref, dis_ref, bi_ref):
    Dv = Dv_ref[0] + Dv_ref[1]
    Be = Be_ref[0] + Be_ref[1]
    dis_ref[...] = _safe_inv_sqrt(Dv)
    bi_ref[...] = _safe_inv(Be)


def _norms(Dv2, Be2):
    return pl.pallas_call(
        _norms_body,
        out_shape=[
            jax.ShapeDtypeStruct((NPAD, LANE_PAD), jnp.float32),
            jax.ShapeDtypeStruct((NPAD, LANE_PAD), jnp.float32),
        ],
    )(Dv2, Be2)


def _matscale_body(h0_ref, h1_ref, dis_ref, Wb_ref, b_ref, o0_ref, o1_ref,
                   *, act):
    xv = jnp.concatenate([h0_ref[...], h1_ref[...]], axis=1)
    if act:
        dis = dis_ref[...]
        sc = jnp.concatenate(
            [jnp.repeat(dis[:, j:j + 1], F, axis=1) for j in range(D)], axis=1)
        xv = xv * sc
        xv = jnp.where(xv > 0, xv, jnp.exp(xv) - 1.0)
    h = xv @ Wb_ref[...] + b_ref[...]
    dis = dis_ref[...]
    sc = jnp.concatenate(
        [jnp.repeat(dis[:, j:j + 1], F, axis=1) for j in range(D)], axis=1)
    h = h * sc
    o0_ref[...] = h[:, :HALF]
    o1_ref[...] = h[:, HALF:]


def _matscale(h0, h1, dis, Wb, b, act):
    blk = BLK
    return pl.pallas_call(
        functools.partial(_matscale_body, act=act),
        grid=(NPAD // blk,),
        in_specs=[
            pl.BlockSpec((blk, HALF), lambda i: (i, 0)),
            pl.BlockSpec((blk, HALF), lambda i: (i, 0)),
            pl.BlockSpec((blk, LANE_PAD), lambda i: (i, 0)),
            pl.BlockSpec((DF, DF), lambda i: (0, 0)),
            pl.BlockSpec((1, DF), lambda i: (0, 0)),
        ],
        out_specs=[
            pl.BlockSpec((blk, HALF), lambda i: (i, 0)),
            pl.BlockSpec((blk, HALF), lambda i: (i, 0)),
        ],
        out_shape=[
            jax.ShapeDtypeStruct((NPAD, HALF), jnp.float32),
            jax.ShapeDtypeStruct((NPAD, HALF), jnp.float32),
        ],
    )(h0, h1, dis, Wb, b.reshape(1, DF))


def _biscale_body(m0_ref, m1_ref, bi_ref, o0_ref, o1_ref):
    bi = bi_ref[...]
    s0 = jnp.concatenate(
        [jnp.repeat(bi[:, j:j + 1], F, axis=1) for j in range(3)], axis=1)
    s1 = jnp.concatenate(
        [jnp.repeat(bi[:, j:j + 1], F, axis=1) for j in range(3, 6)], axis=1)
    o0_ref[...] = m0_ref[...] * s0
    o1_ref[...] = m1_ref[...] * s1


def _biscale(m0, m1, bi):
    blk = BLK
    return pl.pallas_call(
        _biscale_body,
        grid=(NPAD // blk,),
        in_specs=[
            pl.BlockSpec((blk, HALF), lambda i: (i, 0)),
            pl.BlockSpec((blk, HALF), lambda i: (i, 0)),
            pl.BlockSpec((blk, LANE_PAD), lambda i: (i, 0)),
        ],
        out_specs=[
            pl.BlockSpec((blk, HALF), lambda i: (i, 0)),
            pl.BlockSpec((blk, HALF), lambda i: (i, 0)),
        ],
        out_shape=[
            jax.ShapeDtypeStruct((NPAD, HALF), jnp.float32),
            jax.ShapeDtypeStruct((NPAD, HALF), jnp.float32),
        ],
    )(m0, m1, bi)


def _final_body(h0_ref, h1_ref, dis_ref, W2_ref, o_ref):
    xv = jnp.concatenate([h0_ref[...], h1_ref[...]], axis=1)
    dis = dis_ref[...]
    sc = jnp.concatenate(
        [jnp.repeat(dis[:, j:j + 1], F, axis=1) for j in range(D)], axis=1)
    o_ref[...] = (xv * sc) @ W2_ref[...]


def _final(h0, h1, dis, W_lin2):
    blk = BLK
    return pl.pallas_call(
        _final_body,
        grid=(NPAD // blk,),
        in_specs=[
            pl.BlockSpec((blk, HALF), lambda i: (i, 0)),
            pl.BlockSpec((blk, HALF), lambda i: (i, 0)),
            pl.BlockSpec((blk, LANE_PAD), lambda i: (i, 0)),
            pl.BlockSpec((DF, NCLS), lambda i: (0, 0)),
        ],
        out_specs=pl.BlockSpec((blk, NCLS), lambda i: (i, 0)),
        out_shape=jax.ShapeDtypeStruct((NPAD, NCLS), jnp.float32),
    )(h0, h1, dis, W_lin2)


NSC = 2
NTILE = 16
CH = 128
NNZ_PAD = 163840
RPT = NPAD // NTILE

_SC_MESH = plsc.VectorSubcoreMesh(core_axis_name="c", subcore_axis_name="s")
_SC_PARAMS = pltpu.CompilerParams(use_tc_tiling_on_sc=False,
                                  needs_layout_passes=False)


def _splat(vec, idx):
    lanes = lax.iota(jnp.int32, LANE_PAD)
    sel = jnp.where(lanes == idx, vec, 0.0)
    return jnp.sum(sel, axis=0) * jnp.ones((LANE_PAD,), jnp.float32)


def _alpha_deg_body(px_hbm, pe_hbm, row_hbm, col_hbm, z16_hbm,
                    alpha_hbm, dv_hbm, be_hbm,
                    idxr_v, idxc_v, pxb_v, peb_v, sbuf_v,
                    dv_sh, be_sh, sem):
    c = lax.axis_index("c")
    s = lax.axis_index("s")
    wid = s * NSC + c
    pltpu.sync_copy(z16_hbm.at[pl.ds(s * RPT, RPT)],
                    dv_sh.at[pl.ds(s * RPT, RPT)])
    pltpu.sync_copy(z16_hbm.at[pl.ds(s * RPT, RPT)],
                    be_sh.at[pl.ds(s * RPT, RPT)])
    plsc.subcore_barrier()

    ppw = NNZ_PAD // (NSC * NTILE)
    nch = ppw // CH
    lanes = lax.iota(jnp.int32, LANE_PAD)
    lane_mask = jnp.where(lanes < D, 1.0, 0.0)

    def chunk(g, carry):
        k0 = wid * ppw + g * CH
        pltpu.sync_copy(row_hbm.at[pl.ds(k0, CH)], idxr_v)
        pltpu.sync_copy(col_hbm.at[pl.ds(k0, CH)], idxc_v)
        pltpu.async_copy(px_hbm.at[idxr_v], pxb_v, sem).wait()
        pltpu.async_copy(pe_hbm.at[idxc_v], peb_v, sem).wait()

        def pair(p, carry2):
            v = pxb_v[p, :] + peb_v[p, :]
            sg = 1.0 / (1.0 + jnp.exp(-v))
            kvec = lanes * 0 + (k0 + p)
            valid = jnp.where(kvec < NNZ, 1.0, 0.0)
            sbuf_v[p, :] = sg * lane_mask * valid
            return carry2

        lax.fori_loop(0, CH, pair, 0, unroll=2)
        pltpu.sync_copy(sbuf_v, alpha_hbm.at[pl.ds(k0, CH)])
        pltpu.sync_copy(sbuf_v, dv_sh.at[idxr_v], add=True)
        pltpu.sync_copy(sbuf_v, be_sh.at[idxc_v], add=True)
        return carry

    lax.fori_loop(0, nch, chunk, 0)
    plsc.subcore_barrier()
    pltpu.sync_copy(dv_sh.at[pl.ds(s * RPT, RPT)],
                    dv_hbm.at[c].at[pl.ds(s * RPT, RPT)])
    pltpu.sync_copy(be_sh.at[pl.ds(s * RPT, RPT)],
                    be_hbm.at[c].at[pl.ds(s * RPT, RPT)])


def _sc_alpha_degrees(px, pe, rowp, colp, z16):
    f = pl.kernel(
        _alpha_deg_body,
        out_type=[
            jax.ShapeDtypeStruct((NNZ_PAD, LANE_PAD), jnp.float32),
            jax.ShapeDtypeStruct((NSC, NPAD, LANE_PAD), jnp.float32),
            jax.ShapeDtypeStruct((NSC, NPAD, LANE_PAD), jnp.float32),
        ],
        mesh=_SC_MESH,
        compiler_params=_SC_PARAMS,
        scratch_types=[
            pltpu.VMEM((CH,), jnp.int32),
            pltpu.VMEM((CH,), jnp.int32),
            pltpu.VMEM((CH, LANE_PAD), jnp.float32),
            pltpu.VMEM((CH, LANE_PAD), jnp.float32),
            pltpu.VMEM((CH, LANE_PAD), jnp.float32),
            pltpu.VMEM_SHARED((NPAD, LANE_PAD), jnp.float32),
            pltpu.VMEM_SHARED((NPAD, LANE_PAD), jnp.float32),
            pltpu.SemaphoreType.DMA,
        ],
    )
    return f(px, pe, rowp, colp, z16)


QW = 96
QV = QW // LANE_PAD


def _scpass_body(tab_hbm, alpha_hbm, src_hbm, dst_hbm, z_hbm, out_hbm,
                 idxs_v, idxd_v, alpha_v, rows_v, acc_sh, sem, *, half):
    c = lax.axis_index("c")
    s = lax.axis_index("s")
    q = half * 2 + c
    pltpu.sync_copy(z_hbm.at[pl.ds(s * RPT, RPT)],
                    acc_sh.at[pl.ds(s * RPT, RPT)])
    plsc.subcore_barrier()

    ppt = NNZ_PAD // NTILE
    nch = ppt // CH

    def chunk(g, carry):
        k0 = s * ppt + g * CH
        pltpu.sync_copy(src_hbm.at[pl.ds(k0, CH)], idxs_v)
        pltpu.sync_copy(dst_hbm.at[pl.ds(k0, CH)], idxd_v)
        pltpu.sync_copy(alpha_hbm.at[pl.ds(k0, CH)], alpha_v)
        pltpu.async_copy(tab_hbm.at[c].at[idxs_v], rows_v, sem).wait()

        def pair(p, carry2):
            av = alpha_v[p, :]
            for v in range(QV):
                jidx = (6 * q + v) // 4
                bc = _splat(av, jidx)
                off = v * LANE_PAD
                rows_v[p, pl.ds(off, LANE_PAD)] = (
                    rows_v[p, pl.ds(off, LANE_PAD)] * bc)
            return carry2

        lax.fori_loop(0, CH, pair, 0, unroll=2)
        pltpu.sync_copy(rows_v, acc_sh.at[idxd_v], add=True)
        return carry

    lax.fori_loop(0, nch, chunk, 0)
    plsc.subcore_barrier()
    pltpu.sync_copy(acc_sh.at[pl.ds(s * RPT, RPT)],
                    out_hbm.at[c].at[pl.ds(s * RPT, RPT)])


def _sc_pass_quarters(ta, tb, alpha16, src, dst, z96, half):
    tab = jnp.stack([ta, tb])
    f = pl.kernel(
        functools.partial(_scpass_body, half=half),
        out_type=jax.ShapeDtypeStruct((NSC, NPAD, QW), jnp.float32),
        mesh=_SC_MESH,
        compiler_params=_SC_PARAMS,
        scratch_types=[
            pltpu.VMEM((CH,), jnp.int32),
            pltpu.VMEM((CH,), jnp.int32),
            pltpu.VMEM((CH, LANE_PAD), jnp.float32),
            pltpu.VMEM((CH, QW), jnp.float32),
            pltpu.VMEM_SHARED((NPAD, QW), jnp.float32),
            pltpu.SemaphoreType.DMA,
        ],
    )
    out = f(tab, alpha16, src, dst, z96)
    return out[0], out[1]


def _sc_pass(t0, t1, alpha16, src, dst, z96):
    m00, m01 = _sc_pass_quarters(t0[:, :QW], t0[:, QW:], alpha16, src, dst,
                                 z96, half=0)
    m10, m11 = _sc_pass_quarters(t1[:, :QW], t1[:, QW:], alpha16, src, dst,
                                 z96, half=1)
    return (jnp.concatenate([m00, m01], axis=1),
            jnp.concatenate([m10, m11], axis=1))


def kernel(x, edge_index, hyperedge_attr, W_lin, b_lin, W_sheaf, b_sheaf,
           W_conv1, b_conv1, W_conv2, b_conv2, W_lin2):
    pad = jnp.zeros((NNZ_PAD - NNZ,), jnp.int32)
    rowp = jnp.concatenate([edge_index[0], pad])
    colp = jnp.concatenate([edge_index[1], pad])
    z16 = jnp.zeros((NPAD, LANE_PAD), jnp.float32)
    z96 = jnp.zeros((NPAD, QW), jnp.float32)
    rpad = jnp.zeros((NPAD - N, NFEAT), jnp.float32)
    x = jnp.concatenate([x, rpad])
    hyperedge_attr = jnp.concatenate([hyperedge_attr, rpad])

    eye = jnp.eye(D, dtype=jnp.float32)
    W1b = jnp.kron(eye, W_conv1)
    W2b = jnp.kron(eye, W_conv2)
    b1t = jnp.tile(b_conv1, D)
    b2t = jnp.tile(b_conv2, D)

    xs0, xs1, px, pe = _dense1(x, hyperedge_attr, W_lin, b_lin,
                               W_sheaf, b_sheaf)

    alpha16, Dv2, Be2 = _sc_alpha_degrees(px, pe, rowp, colp, z16)
    dis, bi = _norms(Dv2, Be2)

    def conv(v0, v1, Wb, bt, act):
        h0, h1 = _matscale(v0, v1, dis, Wb, bt, act)
        m0, m1 = _sc_pass(h0, h1, alpha16, rowp, colp, z96)
        m0, m1 = _biscale(m0, m1, bi)
        return _sc_pass(m0, m1, alpha16, colp, rowp, z96)

    o0, o1 = conv(xs0, xs1, W1b, b1t, act=False)
    o0, o1 = conv(o0, o1, W2b, b2t, act=True)
    return _final(o0, o1, dis, W_lin2)[:N]


# speedup vs baseline: 7.0983x; 1.8480x over previous
import functools

import jax
import jax.numpy as jnp
import numpy as np
from jax import lax
from jax.experimental import pallas as pl
from jax.experimental.pallas import tpu as pltpu
from jax.experimental.pallas import tpu_sc as plsc

D = 6
F = 64
DF = D * F
HALF = DF // 2
N = 10000
E = 10000
NPAD = 10240
BLK = 1024
NNZ = 160000
NFEAT = 256
NCLS = 40
LANE_PAD = 16


def _safe_inv_sqrt(v):
    vs = jnp.where(v > 0, v, 1.0)
    return jnp.where(v > 0, lax.rsqrt(vs), 0.0)


def _safe_inv(v):
    vs = jnp.where(v > 0, v, 1.0)
    return jnp.where(v > 0, 1.0 / vs, 0.0)


def _dense1_body(x_ref, he_ref, Wl_ref, bl_ref, Ws_ref, bs_ref,
                 xs0_ref, xs1_ref, px_ref, pe_ref):
    Wl = Wl_ref[...]
    bl = bl_ref[...]
    Wm = sum(Wl[:, j * F:(j + 1) * F] for j in range(D)) * (1.0 / D)
    bm = sum(bl[:, j * F:(j + 1) * F] for j in range(D)) * (1.0 / D)
    A0 = Wm @ Ws_ref[:F, :]
    A1 = Wm @ Ws_ref[F:, :]
    b0 = bm @ Ws_ref[:F, :]
    b1 = bm @ Ws_ref[F:, :] + bs_ref[...]

    xl = x_ref[...] @ Wl + bl
    xs0_ref[...] = xl[:, :HALF]
    xs1_ref[...] = xl[:, HALF:]

    pxv = x_ref[...] @ A0 + b0
    pev = he_ref[...] @ A1 + b1
    zeros = jnp.zeros((pxv.shape[0], LANE_PAD - D), jnp.float32)
    px_ref[...] = jnp.concatenate([pxv, zeros], axis=1)
    pe_ref[...] = jnp.concatenate([pev, zeros], axis=1)


def _dense1(x, he, W_lin, b_lin, W_sheaf, b_sheaf):
    blk = BLK
    grid = NPAD // blk
    return pl.pallas_call(
        _dense1_body,
        grid=(grid,),
        in_specs=[
            pl.BlockSpec((blk, NFEAT), lambda i: (i, 0)),
            pl.BlockSpec((blk, NFEAT), lambda i: (i, 0)),
            pl.BlockSpec((NFEAT, DF), lambda i: (0, 0)),
            pl.BlockSpec((1, DF), lambda i: (0, 0)),
            pl.BlockSpec((2 * F, D), lambda i: (0, 0)),
            pl.BlockSpec((1, D), lambda i: (0, 0)),
        ],
        out_specs=[
            pl.BlockSpec((blk, HALF), lambda i: (i, 0)),
            pl.BlockSpec((blk, HALF), lambda i: (i, 0)),
            pl.BlockSpec((blk, LANE_PAD), lambda i: (i, 0)),
            pl.BlockSpec((blk, LANE_PAD), lambda i: (i, 0)),
        ],
        out_shape=[
            jax.ShapeDtypeStruct((NPAD, HALF), jnp.float32),
            jax.ShapeDtypeStruct((NPAD, HALF), jnp.float32),
            jax.ShapeDtypeStruct((NPAD, LANE_PAD), jnp.float32),
            jax.ShapeDtypeStruct((NPAD, LANE_PAD), jnp.float32),
        ],
    )(x, he, W_lin, b_lin.reshape(1, DF), W_sheaf, b_sheaf.reshape(1, D))


def _norms_body(Dv_ref, Be_ref, dis_ref, bi_ref):
    Dv = Dv_ref[0] + Dv_ref[1]
    Be = Be_ref[0] + Be_ref[1]
    dis_ref[...] = _safe_inv_sqrt(Dv)
    bi_ref[...] = _safe_inv(Be)


def _norms(Dv2, Be2):
    return pl.pallas_call(
        _norms_body,
        out_shape=[
            jax.ShapeDtypeStruct((NPAD, LANE_PAD), jnp.float32),
            jax.ShapeDtypeStruct((NPAD, LANE_PAD), jnp.float32),
        ],
    )(Dv2, Be2)


def _matscale_body(h0_ref, h1_ref, dis_ref, Wb_ref, b_ref, o0_ref, o1_ref,
                   *, act):
    xv = jnp.concatenate([h0_ref[...], h1_ref[...]], axis=1)
    if act:
        dis = dis_ref[...]
        sc = jnp.concatenate(
            [jnp.repeat(dis[:, j:j + 1], F, axis=1) for j in range(D)], axis=1)
        xv = xv * sc
        xv = jnp.where(xv > 0, xv, jnp.exp(xv) - 1.0)
    h = xv @ Wb_ref[...] + b_ref[...]
    dis = dis_ref[...]
    sc = jnp.concatenate(
        [jnp.repeat(dis[:, j:j + 1], F, axis=1) for j in range(D)], axis=1)
    h = h * sc
    o0_ref[...] = h[:, :HALF]
    o1_ref[...] = h[:, HALF:]


def _matscale(h0, h1, dis, Wb, b, act):
    blk = BLK
    return pl.pallas_call(
        functools.partial(_matscale_body, act=act),
        grid=(NPAD // blk,),
        in_specs=[
            pl.BlockSpec((blk, HALF), lambda i: (i, 0)),
            pl.BlockSpec((blk, HALF), lambda i: (i, 0)),
            pl.BlockSpec((blk, LANE_PAD), lambda i: (i, 0)),
            pl.BlockSpec((DF, DF), lambda i: (0, 0)),
            pl.BlockSpec((1, DF), lambda i: (0, 0)),
        ],
        out_specs=[
            pl.BlockSpec((blk, HALF), lambda i: (i, 0)),
            pl.BlockSpec((blk, HALF), lambda i: (i, 0)),
        ],
        out_shape=[
            jax.ShapeDtypeStruct((NPAD, HALF), jnp.float32),
            jax.ShapeDtypeStruct((NPAD, HALF), jnp.float32),
        ],
    )(h0, h1, dis, Wb, b.reshape(1, DF))


def _biscale_body(m0_ref, m1_ref, bi_ref, o0_ref, o1_ref):
    bi = bi_ref[...]
    s0 = jnp.concatenate(
        [jnp.repeat(bi[:, j:j + 1], F, axis=1) for j in range(3)], axis=1)
    s1 = jnp.concatenate(
        [jnp.repeat(bi[:, j:j + 1], F, axis=1) for j in range(3, 6)], axis=1)
    o0_ref[...] = m0_ref[...] * s0
    o1_ref[...] = m1_ref[...] * s1


def _biscale(m0, m1, bi):
    blk = BLK
    return pl.pallas_call(
        _biscale_body,
        grid=(NPAD // blk,),
        in_specs=[
            pl.BlockSpec((blk, HALF), lambda i: (i, 0)),
            pl.BlockSpec((blk, HALF), lambda i: (i, 0)),
            pl.BlockSpec((blk, LANE_PAD), lambda i: (i, 0)),
        ],
        out_specs=[
            pl.BlockSpec((blk, HALF), lambda i: (i, 0)),
            pl.BlockSpec((blk, HALF), lambda i: (i, 0)),
        ],
        out_shape=[
            jax.ShapeDtypeStruct((NPAD, HALF), jnp.float32),
            jax.ShapeDtypeStruct((NPAD, HALF), jnp.float32),
        ],
    )(m0, m1, bi)


def _final_body(h0_ref, h1_ref, dis_ref, W2_ref, o_ref):
    xv = jnp.concatenate([h0_ref[...], h1_ref[...]], axis=1)
    dis = dis_ref[...]
    sc = jnp.concatenate(
        [jnp.repeat(dis[:, j:j + 1], F, axis=1) for j in range(D)], axis=1)
    o_ref[...] = (xv * sc) @ W2_ref[...]


def _final(h0, h1, dis, W_lin2):
    blk = BLK
    return pl.pallas_call(
        _final_body,
        grid=(NPAD // blk,),
        in_specs=[
            pl.BlockSpec((blk, HALF), lambda i: (i, 0)),
            pl.BlockSpec((blk, HALF), lambda i: (i, 0)),
            pl.BlockSpec((blk, LANE_PAD), lambda i: (i, 0)),
            pl.BlockSpec((DF, NCLS), lambda i: (0, 0)),
        ],
        out_specs=pl.BlockSpec((blk, NCLS), lambda i: (i, 0)),
        out_shape=jax.ShapeDtypeStruct((NPAD, NCLS), jnp.float32),
    )(h0, h1, dis, W_lin2)


NSC = 2
NTILE = 16
CH = 128
NNZ_PAD = 163840
RPT = NPAD // NTILE

_SC_MESH = plsc.VectorSubcoreMesh(core_axis_name="c", subcore_axis_name="s")
_SC_PARAMS = pltpu.CompilerParams(use_tc_tiling_on_sc=False,
                                  needs_layout_passes=False)


def _splat(vec, idx):
    lanes = lax.iota(jnp.int32, LANE_PAD)
    sel = jnp.where(lanes == idx, vec, 0.0)
    return jnp.sum(sel, axis=0) * jnp.ones((LANE_PAD,), jnp.float32)


NCHW = NNZ_PAD // (NSC * NTILE) // CH


def _alpha_deg_body(px_hbm, pe_hbm, row2_hbm, col2_hbm, z16_hbm,
                    alpha_hbm, dv_hbm, be_hbm,
                    idxr_v, idxc_v, px0_v, px1_v, pe0_v, pe1_v,
                    sb0_v, sb1_v, dv_sh, be_sh, xs0, xs1, es0, es1):
    c = lax.axis_index("c")
    s = lax.axis_index("s")
    wid = s * NSC + c
    pltpu.sync_copy(z16_hbm.at[pl.ds(s * RPT, RPT)],
                    dv_sh.at[pl.ds(s * RPT, RPT)])
    pltpu.sync_copy(z16_hbm.at[pl.ds(s * RPT, RPT)],
                    be_sh.at[pl.ds(s * RPT, RPT)])
    pltpu.sync_copy(row2_hbm.at[pl.ds(wid * NCHW, NCHW)], idxr_v)
    pltpu.sync_copy(col2_hbm.at[pl.ds(wid * NCHW, NCHW)], idxc_v)
    plsc.subcore_barrier()

    lanes = lax.iota(jnp.int32, LANE_PAD)
    lane_mask = jnp.where(lanes < D, 1.0, 0.0)
    bufs = ((px0_v, pe0_v, sb0_v, xs0, es0), (px1_v, pe1_v, sb1_v, xs1, es1))

    def fetch(g, b):
        pxb, peb, sb, xsem, esem = bufs[b]
        pltpu.async_copy(px_hbm.at[idxr_v.at[g]], pxb, xsem)
        pltpu.async_copy(pe_hbm.at[idxc_v.at[g]], peb, esem)

    def step(g, b):
        pxb, peb, sb, xsem, esem = bufs[b]
        pltpu.make_async_copy(px_hbm.at[idxr_v.at[0]], pxb, xsem).wait()
        pltpu.make_async_copy(pe_hbm.at[idxc_v.at[0]], peb, esem).wait()
        k0 = (wid * NCHW + g) * CH

        def pair(p, carry2):
            v = pxb[p, :] + peb[p, :]
            sg = 1.0 / (1.0 + jnp.exp(-v))
            kvec = lanes * 0 + (k0 + p)
            valid = jnp.where(kvec < NNZ, 1.0, 0.0)
            sb[p, :] = sg * lane_mask * valid
            return carry2

        lax.fori_loop(0, CH, pair, 0, unroll=4)
        pltpu.sync_copy(sb, alpha_hbm.at[pl.ds(k0, CH)])
        pltpu.sync_copy(sb, dv_sh.at[idxr_v.at[g]], add=True)
        pltpu.sync_copy(sb, be_sh.at[idxc_v.at[g]], add=True)

    fetch(0, 0)
    fetch(1, 1)

    def outer(g2, carry):
        for b in range(2):
            g = g2 * 2 + b
            step(g, b)

            @pl.when(g + 2 < NCHW)
            def _():
                fetch(g + 2, b)
        return carry

    lax.fori_loop(0, NCHW // 2, outer, 0)
    plsc.subcore_barrier()
    pltpu.sync_copy(dv_sh.at[pl.ds(s * RPT, RPT)],
                    dv_hbm.at[c].at[pl.ds(s * RPT, RPT)])
    pltpu.sync_copy(be_sh.at[pl.ds(s * RPT, RPT)],
                    be_hbm.at[c].at[pl.ds(s * RPT, RPT)])


def _sc_alpha_degrees(px, pe, row2, col2, z16):
    f = pl.kernel(
        _alpha_deg_body,
        out_type=[
            jax.ShapeDtypeStruct((NNZ_PAD, LANE_PAD), jnp.float32),
            jax.ShapeDtypeStruct((NSC, NPAD, LANE_PAD), jnp.float32),
            jax.ShapeDtypeStruct((NSC, NPAD, LANE_PAD), jnp.float32),
        ],
        mesh=_SC_MESH,
        compiler_params=_SC_PARAMS,
        scratch_types=[
            pltpu.VMEM((NCHW, CH), jnp.int32),
            pltpu.VMEM((NCHW, CH), jnp.int32),
            pltpu.VMEM((CH, LANE_PAD), jnp.float32),
            pltpu.VMEM((CH, LANE_PAD), jnp.float32),
            pltpu.VMEM((CH, LANE_PAD), jnp.float32),
            pltpu.VMEM((CH, LANE_PAD), jnp.float32),
            pltpu.VMEM((CH, LANE_PAD), jnp.float32),
            pltpu.VMEM((CH, LANE_PAD), jnp.float32),
            pltpu.VMEM_SHARED((NPAD, LANE_PAD), jnp.float32),
            pltpu.VMEM_SHARED((NPAD, LANE_PAD), jnp.float32),
            pltpu.SemaphoreType.DMA,
            pltpu.SemaphoreType.DMA,
            pltpu.SemaphoreType.DMA,
            pltpu.SemaphoreType.DMA,
        ],
    )
    return f(px, pe, row2, col2, z16)


QW = 96
QV = QW // LANE_PAD


NCHT = NNZ_PAD // NTILE // CH


def _scpass_body(tab_hbm, alpha_hbm, src2_hbm, dst2_hbm, z_hbm, out_hbm,
                 idxs_v, idxd_v, al0_v, al1_v, rw0_v, rw1_v, acc_sh,
                 gs0, gs1, as0, as1, *, half):
    c = lax.axis_index("c")
    s = lax.axis_index("s")
    q = half * 2 + c
    pltpu.sync_copy(z_hbm.at[pl.ds(s * RPT, RPT)],
                    acc_sh.at[pl.ds(s * RPT, RPT)])
    pltpu.sync_copy(src2_hbm.at[pl.ds(s * NCHT, NCHT)], idxs_v)
    pltpu.sync_copy(dst2_hbm.at[pl.ds(s * NCHT, NCHT)], idxd_v)
    plsc.subcore_barrier()

    bufs = ((rw0_v, al0_v, gs0, as0), (rw1_v, al1_v, gs1, as1))
    lanes = lax.iota(jnp.int32, LANE_PAD)
    ivs = [lanes * 0 + (6 * q + v) // 4 for v in range(QV)]

    def fetch(g, b):
        rows, al, gsem, asem = bufs[b]
        pltpu.async_copy(tab_hbm.at[c].at[idxs_v.at[g]], rows, gsem)
        pltpu.async_copy(alpha_hbm.at[pl.ds((s * NCHT + g) * CH, CH)],
                         al, asem)

    def step(g, b):
        rows, al, gsem, asem = bufs[b]
        pltpu.make_async_copy(tab_hbm.at[c].at[idxs_v.at[0]],
                              rows, gsem).wait()
        pltpu.make_async_copy(alpha_hbm.at[pl.ds(0, CH)], al, asem).wait()

        def pair(p, carry2):
            av = al[p, :]
            for v in range(QV):
                bc = av.at[ivs[v]].get(mode="promise_in_bounds")
                off = v * LANE_PAD
                rows[p, pl.ds(off, LANE_PAD)] = (
                    rows[p, pl.ds(off, LANE_PAD)] * bc)
            return carry2

        lax.fori_loop(0, CH, pair, 0, unroll=4)
        pltpu.sync_copy(rows, acc_sh.at[idxd_v.at[g]], add=True)

    fetch(0, 0)
    fetch(1, 1)

    def outer(g2, carry):
        for b in range(2):
            g = g2 * 2 + b
            step(g, b)

            @pl.when(g + 2 < NCHT)
            def _():
                fetch(g + 2, b)
        return carry

    lax.fori_loop(0, NCHT // 2, outer, 0)
    plsc.subcore_barrier()
    pltpu.sync_copy(acc_sh.at[pl.ds(s * RPT, RPT)],
                    out_hbm.at[c].at[pl.ds(s * RPT, RPT)])


def _sc_pass_quarters(ta, tb, alpha16, src2, dst2, z96, half):
    tab = jnp.stack([ta, tb])
    f = pl.kernel(
        functools.partial(_scpass_body, half=half),
        out_type=jax.ShapeDtypeStruct((NSC, NPAD, QW), jnp.float32),
        mesh=_SC_MESH,
        compiler_params=_SC_PARAMS,
        scratch_types=[
            pltpu.VMEM((NCHT, CH), jnp.int32),
            pltpu.VMEM((NCHT, CH), jnp.int32),
            pltpu.VMEM((CH, LANE_PAD), jnp.float32),
            pltpu.VMEM((CH, LANE_PAD), jnp.float32),
            pltpu.VMEM((CH, QW), jnp.float32),
            pltpu.VMEM((CH, QW), jnp.float32),
            pltpu.VMEM_SHARED((NPAD, QW), jnp.float32),
            pltpu.SemaphoreType.DMA,
            pltpu.SemaphoreType.DMA,
            pltpu.SemaphoreType.DMA,
            pltpu.SemaphoreType.DMA,
        ],
    )
    out = f(tab, alpha16, src2, dst2, z96)
    return out[0], out[1]


def _sc_pass(t0, t1, alpha16, src2, dst2, z96):
    m00, m01 = _sc_pass_quarters(t0[:, :QW], t0[:, QW:], alpha16, src2, dst2,
                                 z96, half=0)
    m10, m11 = _sc_pass_quarters(t1[:, :QW], t1[:, QW:], alpha16, src2, dst2,
                                 z96, half=1)
    return (jnp.concatenate([m00, m01], axis=1),
            jnp.concatenate([m10, m11], axis=1))


def kernel(x, edge_index, hyperedge_attr, W_lin, b_lin, W_sheaf, b_sheaf,
           W_conv1, b_conv1, W_conv2, b_conv2, W_lin2):
    pad = jnp.zeros((NNZ_PAD - NNZ,), jnp.int32)
    rowp = jnp.concatenate([edge_index[0], pad]).reshape(NNZ_PAD // CH, CH)
    colp = jnp.concatenate([edge_index[1], pad]).reshape(NNZ_PAD // CH, CH)
    z16 = jnp.zeros((NPAD, LANE_PAD), jnp.float32)
    z96 = jnp.zeros((NPAD, QW), jnp.float32)
    rpad = jnp.zeros((NPAD - N, NFEAT), jnp.float32)
    x = jnp.concatenate([x, rpad])
    hyperedge_attr = jnp.concatenate([hyperedge_attr, rpad])

    eye = jnp.eye(D, dtype=jnp.float32)
    W1b = jnp.kron(eye, W_conv1)
    W2b = jnp.kron(eye, W_conv2)
    b1t = jnp.tile(b_conv1, D)
    b2t = jnp.tile(b_conv2, D)

    xs0, xs1, px, pe = _dense1(x, hyperedge_attr, W_lin, b_lin,
                               W_sheaf, b_sheaf)

    alpha16, Dv2, Be2 = _sc_alpha_degrees(px, pe, rowp, colp, z16)
    dis, bi = _norms(Dv2, Be2)

    def conv(v0, v1, Wb, bt, act):
        h0, h1 = _matscale(v0, v1, dis, Wb, bt, act)
        m0, m1 = _sc_pass(h0, h1, alpha16, rowp, colp, z96)
        m0, m1 = _biscale(m0, m1, bi)
        return _sc_pass(m0, m1, alpha16, colp, rowp, z96)

    o0, o1 = conv(xs0, xs1, W1b, b1t, act=False)
    o0, o1 = conv(o0, o1, W2b, b2t, act=True)
    return _final(o0, o1, dis, W_lin2)[:N]
